# baseline (speedup 1.0000x reference)
import functools

import jax
import jax.numpy as jnp
from jax import lax
from jax.experimental import pallas as pl
from jax.experimental.pallas import tpu as pltpu

_BM = 1000
_NBUF = 5


def _make_body(n, d_in, d_hid):
    nch = n // _BM
    ng = nch // _NBUF

    def body(x_hbm, w1_ref, b1_ref, w2_ref, b2_ref, o_hbm,
             xbuf, obuf, insem, outsem):
        w1 = w1_ref[...]
        b1 = b1_ref[...]
        w2 = w2_ref[...]
        b2 = b2_ref[...]

        def in_copy(ch, slot):
            return pltpu.make_async_copy(
                x_hbm.at[pl.ds(ch * _BM, _BM), :], xbuf.at[slot],
                insem.at[slot])

        def out_copy(ch, slot,
                     ):
            return pltpu.make_async_copy(
                obuf.at[slot], o_hbm.at[pl.ds(ch * _BM, _BM), :],
                outsem.at[slot])

        for s in range(_NBUF):
            in_copy(s, s).start()

        def outer(g, carry):
            for b in range(_NBUF):
                ch = g * _NBUF + b
                in_copy(ch, b).wait()
                xb = xbuf[b].astype(jnp.bfloat16)
                h = lax.dot_general(
                    xb, w1, (((1,), (1,)), ((), ())),
                    preferred_element_type=jnp.float32,
                )
                h = jnp.maximum(h + b1, 0.0).astype(jnp.bfloat16)
                o = lax.dot_general(
                    h, w2, (((1,), (1,)), ((), ())),
                    preferred_element_type=jnp.float32,
                )

                @pl.when(g > 0)
                def _():
                    out_copy(ch, b).wait()

                obuf[b] = o + b2
                out_copy(ch, b).start()

                @pl.when(ch + _NBUF < nch)
                def _():
                    in_copy(ch + _NBUF, b).start()
            return carry

        lax.fori_loop(0, ng, outer, 0)

        for s in range(_NBUF):
            out_copy(0, s).wait()

    return body


@jax.jit
def _fused_mlp(x, W1, b1, W2, b2):
    n, d_in = x.shape
    d_hid = W1.shape[0]
    return pl.pallas_call(
        _make_body(n, d_in, d_hid),
        in_specs=[
            pl.BlockSpec(memory_space=pl.ANY),
            pl.BlockSpec(memory_space=pltpu.MemorySpace.VMEM),
            pl.BlockSpec(memory_space=pltpu.MemorySpace.VMEM),
            pl.BlockSpec(memory_space=pltpu.MemorySpace.VMEM),
            pl.BlockSpec(memory_space=pltpu.MemorySpace.VMEM),
        ],
        out_specs=pl.BlockSpec(memory_space=pl.ANY),
        out_shape=jax.ShapeDtypeStruct((n, d_hid), jnp.float32),
        scratch_shapes=[
            pltpu.VMEM((_NBUF, _BM, d_in), jnp.float32),
            pltpu.VMEM((_NBUF, _BM, d_hid), jnp.float32),
            pltpu.SemaphoreType.DMA((_NBUF,)),
            pltpu.SemaphoreType.DMA((_NBUF,)),
        ],
        compiler_params=pltpu.CompilerParams(
            vmem_limit_bytes=100 * 1024 * 1024,
        ),
    )(x, W1.astype(jnp.bfloat16), b1.reshape(1, -1),
      W2.astype(jnp.bfloat16), b2.reshape(1, -1))


def kernel(x, W1, b1, W2, b2):
    return _fused_mlp(x, W1, b1, W2, b2)


# speedup vs baseline: 1.1788x; 1.1788x over previous
import functools

import jax
import jax.numpy as jnp
from jax import lax
from jax.experimental import pallas as pl
from jax.experimental.pallas import tpu as pltpu

_BM = 2000
_NBUF = 5


def _make_body(n, d_in, d_hid):
    nch = n // _BM
    ng = nch // _NBUF

    def body(x_hbm, w1_ref, b1_ref, w2_ref, b2_ref, o_hbm,
             xbuf, obuf, insem, outsem):
        w1 = w1_ref[...]
        b1 = b1_ref[...]
        w2 = w2_ref[...]
        b2 = b2_ref[...]

        def in_copy(ch, slot):
            return pltpu.make_async_copy(
                x_hbm.at[pl.ds(ch * _BM, _BM), :], xbuf.at[slot],
                insem.at[slot])

        def out_copy(ch, slot,
                     ):
            return pltpu.make_async_copy(
                obuf.at[slot], o_hbm.at[pl.ds(ch * _BM, _BM), :],
                outsem.at[slot])

        for s in range(_NBUF):
            in_copy(s, s).start()

        def outer(g, carry):
            for b in range(_NBUF):
                ch = g * _NBUF + b
                in_copy(ch, b).wait()
                xb = xbuf[b].astype(jnp.bfloat16)
                h = lax.dot_general(
                    xb, w1, (((1,), (1,)), ((), ())),
                    preferred_element_type=jnp.float32,
                )
                h = jnp.maximum(h + b1, 0.0).astype(jnp.bfloat16)
                o = lax.dot_general(
                    h, w2, (((1,), (1,)), ((), ())),
                    preferred_element_type=jnp.float32,
                )

                @pl.when(g > 0)
                def _():
                    out_copy(ch, b).wait()

                obuf[b] = o + b2
                out_copy(ch, b).start()

                @pl.when(ch + _NBUF < nch)
                def _():
                    in_copy(ch + _NBUF, b).start()
            return carry

        lax.fori_loop(0, ng, outer, 0)

        for s in range(_NBUF):
            out_copy(0, s).wait()

    return body


@jax.jit
def _fused_mlp(x, W1, b1, W2, b2):
    n, d_in = x.shape
    d_hid = W1.shape[0]
    return pl.pallas_call(
        _make_body(n, d_in, d_hid),
        in_specs=[
            pl.BlockSpec(memory_space=pl.ANY),
            pl.BlockSpec(memory_space=pltpu.MemorySpace.VMEM),
            pl.BlockSpec(memory_space=pltpu.MemorySpace.VMEM),
            pl.BlockSpec(memory_space=pltpu.MemorySpace.VMEM),
            pl.BlockSpec(memory_space=pltpu.MemorySpace.VMEM),
        ],
        out_specs=pl.BlockSpec(memory_space=pl.ANY),
        out_shape=jax.ShapeDtypeStruct((n, d_hid), jnp.float32),
        scratch_shapes=[
            pltpu.VMEM((_NBUF, _BM, d_in), jnp.float32),
            pltpu.VMEM((_NBUF, _BM, d_hid), jnp.float32),
            pltpu.SemaphoreType.DMA((_NBUF,)),
            pltpu.SemaphoreType.DMA((_NBUF,)),
        ],
        compiler_params=pltpu.CompilerParams(
            vmem_limit_bytes=100 * 1024 * 1024,
        ),
    )(x, W1.astype(jnp.bfloat16), b1.reshape(1, -1),
      W2.astype(jnp.bfloat16), b2.reshape(1, -1))


def kernel(x, W1, b1, W2, b2):
    return _fused_mlp(x, W1, b1, W2, b2)


# speedup vs baseline: 1.2496x; 1.0601x over previous
import functools

import jax
import jax.numpy as jnp
from jax import lax
from jax.experimental import pallas as pl
from jax.experimental.pallas import tpu as pltpu


def _mlp_kernel(x_ref, w1_ref, b1_ref, w2_ref, b2_ref, o_ref,
                w1b_ref, w2b_ref):
    @pl.when(pl.program_id(0) == 0)
    def _():
        w1b_ref[...] = w1_ref[...].astype(jnp.bfloat16)
        w2b_ref[...] = w2_ref[...].astype(jnp.bfloat16)

    x = x_ref[...].astype(jnp.bfloat16)
    h = lax.dot_general(
        x, w1b_ref[...], (((1,), (1,)), ((), ())),
        preferred_element_type=jnp.float32,
    )
    h = jnp.maximum(h + b1_ref[...], 0.0).astype(jnp.bfloat16)
    o = lax.dot_general(
        h, w2b_ref[...], (((1,), (1,)), ((), ())),
        preferred_element_type=jnp.float32,
    )
    o_ref[...] = o + b2_ref[...]


@functools.partial(jax.jit, static_argnames=("block_m",))
def _fused_mlp(x, W1, b1, W2, b2, block_m):
    n, d_in = x.shape
    d_hid = W1.shape[0]
    grid = (pl.cdiv(n, block_m),)
    return pl.pallas_call(
        _mlp_kernel,
        grid=grid,
        in_specs=[
            pl.BlockSpec((block_m, d_in), lambda i: (i, 0)),
            pl.BlockSpec((d_hid, d_in), lambda i: (0, 0)),
            pl.BlockSpec((1, d_hid), lambda i: (0, 0)),
            pl.BlockSpec((d_hid, d_hid), lambda i: (0, 0)),
            pl.BlockSpec((1, d_hid), lambda i: (0, 0)),
        ],
        out_specs=pl.BlockSpec((block_m, d_hid), lambda i: (i, 0)),
        out_shape=jax.ShapeDtypeStruct((n, d_hid), jnp.float32),
        scratch_shapes=[
            pltpu.VMEM((d_hid, d_in), jnp.bfloat16),
            pltpu.VMEM((d_hid, d_hid), jnp.bfloat16),
        ],
        compiler_params=pltpu.CompilerParams(
            dimension_semantics=("arbitrary",),
            vmem_limit_bytes=100 * 1024 * 1024,
        ),
    )(x, W1, b1.reshape(1, -1), W2, b2.reshape(1, -1))


def kernel(x, W1, b1, W2, b2):
    return _fused_mlp(x, W1, b1, W2, b2, block_m=5000)
